# baseline (speedup 1.0000x reference)
import functools
import jax
import jax.numpy as jnp
from jax import lax
from jax.experimental import pallas as pl
from jax.experimental.pallas import tpu as pltpu
from jax.experimental.pallas import tpu_sc as plsc


def _make_sc_bcast(S, N, E):
    info = plsc.get_sparse_core_info()
    nw = info.num_cores * info.num_subcores
    rows_per_w = S // nw
    mesh = plsc.VectorSubcoreMesh(core_axis_name="c", subcore_axis_name="s")

    @functools.partial(
        pl.kernel,
        out_type=jax.ShapeDtypeStruct((S, N, E), jnp.float32),
        mesh=mesh,
        scratch_types=[
            pltpu.SemaphoreType.DMA,
        ],
    )
    def sc_bcast(table_hbm, out_hbm, sem):
        wid = lax.axis_index("s") * info.num_cores + lax.axis_index("c")
        r0 = wid * rows_per_w
        copies = [
            pltpu.async_copy(
                table_hbm.at[pl.ds(r0, rows_per_w)],
                out_hbm.at[pl.ds(r0, rows_per_w), n],
                sem,
            )
            for n in range(N)
        ]
        for c in copies:
            c.wait()

    return sc_bcast


def kernel(x, table):
    S, N = x.shape
    _, E = table.shape
    return _make_sc_bcast(S, N, E)(table)


# speedup vs baseline: 13.9740x; 13.9740x over previous
import functools
import jax
import jax.numpy as jnp
from jax import lax
from jax.experimental import pallas as pl
from jax.experimental.pallas import tpu as pltpu
from jax.experimental.pallas import tpu_sc as plsc


def _make_sc_bcast(S_sc, N, E, row_off):
    info = plsc.get_sparse_core_info()
    nw = info.num_cores * info.num_subcores
    rows_per_w = S_sc // nw
    mesh = plsc.VectorSubcoreMesh(core_axis_name="c", subcore_axis_name="s")

    @functools.partial(
        pl.kernel,
        out_type=jax.ShapeDtypeStruct((S_sc, N, E), jnp.float32),
        mesh=mesh,
        scratch_types=[
            pltpu.VMEM((rows_per_w, E), jnp.float32),
            pltpu.SemaphoreType.DMA,
        ],
    )
    def sc_bcast(table_hbm, out_hbm, buf, sem):
        wid = lax.axis_index("s") * info.num_cores + lax.axis_index("c")
        r0 = wid * rows_per_w
        pltpu.sync_copy(table_hbm.at[pl.ds(row_off + r0, rows_per_w)], buf)
        copies = [
            pltpu.async_copy(buf, out_hbm.at[pl.ds(r0, rows_per_w), n], sem)
            for n in range(N)
        ]
        for c in copies:
            c.wait()

    return sc_bcast


def _tc_body(t_ref, o_ref):
    o_ref[...] = jnp.broadcast_to(
        t_ref[...][:, None, :], (o_ref.shape[0],) + o_ref.shape[1:]
    )


def _tc_bcast(table, S_tc, N, E, block_s):
    return pl.pallas_call(
        _tc_body,
        grid=(S_tc // block_s,),
        in_specs=[pl.BlockSpec((block_s, E), lambda i: (i, 0))],
        out_specs=pl.BlockSpec((block_s, N, E), lambda i: (i, 0, 0)),
        out_shape=jax.ShapeDtypeStruct((S_tc, N, E), table.dtype),
    )(table)


def kernel(x, table):
    S, N = x.shape
    _, E = table.shape
    S_tc = S // 2
    S_sc = S - S_tc
    out_tc = _tc_bcast(table[:S_tc], S_tc, N, E, 256)
    out_sc = _make_sc_bcast(S_sc, N, E, S_tc)(table)
    return jnp.concatenate([out_tc, out_sc], axis=0)


# speedup vs baseline: 31.5938x; 2.2609x over previous
import functools
import jax
import jax.numpy as jnp
from jax import lax
from jax.experimental import pallas as pl
from jax.experimental.pallas import tpu as pltpu
from jax.experimental.pallas import tpu_sc as plsc


def _make_sc_bcast(S, N, E):
    info = plsc.get_sparse_core_info()
    nw = info.num_cores * info.num_subcores
    rows_per_w = S // nw
    mesh = plsc.VectorSubcoreMesh(core_axis_name="c", subcore_axis_name="s")

    @functools.partial(
        pl.kernel,
        out_type=jax.ShapeDtypeStruct((S, N, E), jnp.float32),
        mesh=mesh,
        scratch_types=[
            pltpu.VMEM((rows_per_w, E), jnp.float32),
            pltpu.SemaphoreType.DMA,
            pltpu.SemaphoreType.DMA,
        ],
    )
    def sc_bcast(table_hbm, out_hbm, buf, sem_r, sem_w):
        wid = lax.axis_index("s") * info.num_cores + lax.axis_index("c")
        r0 = wid * rows_per_w
        ch = rows_per_w // 2
        reads = [
            pltpu.async_copy(
                table_hbm.at[pl.ds(r0 + k * ch, ch)], buf.at[pl.ds(k * ch, ch)], sem_r
            )
            for k in range(2)
        ]
        writes = []
        for k in range(2):
            reads[k].wait()
            for n in range(N):
                writes.append(
                    pltpu.async_copy(
                        buf.at[pl.ds(k * ch, ch)],
                        out_hbm.at[pl.ds(r0 + k * ch, ch), n],
                        sem_w,
                    )
                )
        for w in writes:
            w.wait()

    return sc_bcast


def kernel(x, table):
    S, N = x.shape
    _, E = table.shape
    return _make_sc_bcast(S, N, E)(table)
